# baseline (speedup 1.0000x reference)
import functools
import math

import jax
import jax.numpy as jnp
from jax import lax
from jax.experimental import pallas as pl
from jax.experimental.pallas import tpu as pltpu
from jax.experimental.pallas import tpu_sc as plsc

B, L, D, E, FF = 2, 2048, 1024, 16, 4096
N = B * L
CAP = N // E
EPS = 1e-05

TOK_BLK = 512
FF_BLK = 1024


def _router_body(x_ref, nw_ref, rw_ref, rb_ref, xn_ref, kt_ref, pt_ref):
    x = x_ref[...]
    var = jnp.mean(x * x, axis=1, keepdims=True)
    xn = x * lax.rsqrt(var + EPS) * nw_ref[...]
    xn_ref[...] = xn.astype(jnp.bfloat16)
    lt = lax.dot_general(rw_ref[...], xn, (((1,), (1,)), ((), ())),
                         preferred_element_type=jnp.float32)
    lt = lt + rb_ref[...].reshape(E, 1)
    kb = lax.bitcast_convert_type(lt, jnp.int32)
    kt_ref[...] = jnp.where(kb < 0, kb ^ jnp.int32(0x7FFFFFFF), kb)
    m = jnp.max(lt, axis=0, keepdims=True)
    ex = jnp.exp(lt - m)
    pt_ref[...] = ex / jnp.sum(ex, axis=0, keepdims=True)


def _router(x_flat, norm_weight, router_weight, router_bias):
    grid = (N // TOK_BLK,)
    return pl.pallas_call(
        _router_body,
        grid=grid,
        in_specs=[
            pl.BlockSpec((TOK_BLK, D), lambda i: (i, 0)),
            pl.BlockSpec((1, D), lambda i: (0, 0)),
            pl.BlockSpec((E, D), lambda i: (0, 0)),
            pl.BlockSpec((1, E), lambda i: (0, 0)),
        ],
        out_specs=[
            pl.BlockSpec((TOK_BLK, D), lambda i: (i, 0)),
            pl.BlockSpec((E, TOK_BLK), lambda i: (0, i)),
            pl.BlockSpec((E, TOK_BLK), lambda i: (0, i)),
        ],
        out_shape=[
            jax.ShapeDtypeStruct((N, D), jnp.bfloat16),
            jax.ShapeDtypeStruct((E, N), jnp.int32),
            jax.ShapeDtypeStruct((E, N), jnp.float32),
        ],
    )(x_flat, norm_weight.reshape(1, D), router_weight, router_bias.reshape(1, E))


def _gelu_exact(h):
    return 0.5 * h * (1.0 + lax.erf(h * (1.0 / math.sqrt(2.0))))


def _mlp_body(xg_ref, g_ref, w1_ref, b1_ref, w2_ref, b2_ref, out_ref, acc_ref):
    f = pl.program_id(1)
    xg = xg_ref[0]
    w1 = w1_ref[0].astype(jnp.bfloat16)
    h = lax.dot_general(xg, w1, (((1,), (1,)), ((), ())),
                        preferred_element_type=jnp.float32)
    h = h + b1_ref[0]
    h = _gelu_exact(h)
    w2 = w2_ref[0].astype(jnp.bfloat16)
    y = lax.dot_general(h.astype(jnp.bfloat16), w2, (((1,), (1,)), ((), ())),
                        preferred_element_type=jnp.float32)

    @pl.when(f == 0)
    def _():
        acc_ref[...] = y + b2_ref[0]

    @pl.when(f > 0)
    def _():
        acc_ref[...] += y

    @pl.when(f == FF // FF_BLK - 1)
    def _():
        out_ref[0] = acc_ref[...] * g_ref[0].reshape(CAP, 1)


def _mlp(xg_g, g_norm, fc1_weight, fc1_bias, fc2_weight, fc2_bias, e0, eg):
    grid = (eg, FF // FF_BLK)
    return pl.pallas_call(
        _mlp_body,
        grid=grid,
        in_specs=[
            pl.BlockSpec((1, CAP, D), lambda e, f: (e, 0, 0)),
            pl.BlockSpec((1, 1, CAP), lambda e, f: (e + e0, 0, 0)),
            pl.BlockSpec((1, FF_BLK, D), lambda e, f: (e + e0, f, 0)),
            pl.BlockSpec((1, 1, FF_BLK), lambda e, f: (e + e0, 0, f)),
            pl.BlockSpec((1, D, FF_BLK), lambda e, f: (e + e0, 0, f)),
            pl.BlockSpec((1, 1, D), lambda e, f: (e + e0, 0, 0)),
        ],
        out_specs=pl.BlockSpec((1, CAP, D), lambda e, f: (e, 0, 0)),
        out_shape=jax.ShapeDtypeStruct((eg, CAP, D), jnp.float32),
        scratch_shapes=[pltpu.VMEM((CAP, D), jnp.float32)],
        compiler_params=pltpu.CompilerParams(
            dimension_semantics=("parallel", "arbitrary")),
    )(xg_g.reshape(eg, CAP, D), g_norm.reshape(E, 1, CAP),
      fc1_weight, fc1_bias.reshape(E, 1, FF),
      fc2_weight, fc2_bias.reshape(E, 1, D))


NC, NS = 2, 16
NW = NC * NS
KV = N // 16
_MESH = dict(core_axis_name="c", subcore_axis_name="s")
IMIN = -2147483648


def _routing_body(kt_hbm, pt_hbm, tok_hbm, gn_hbm,
                  keys_v, probs_v, gd_v, seg_v, gsp_v, gs_v,
                  idxl_v, gvals_v, gn_v, gs_all_sh, gs_tot_sh):
    cid = lax.axis_index("c")
    sid = lax.axis_index("s")

    @pl.when(cid == 0)
    def _():
        e = sid
        pltpu.sync_copy(kt_hbm.at[e], keys_v)
        pltpu.sync_copy(pt_hbm.at[e], probs_v)

        def lane_sum(v):
            s = v[0]
            for l in range(1, 16):
                s = s + v[l]
            return s

        def popcnt(m):
            return lane_sum(jnp.where(m, jnp.int32(1), jnp.int32(0)))

        def count_ge(ts):
            def cbody(i, acc):
                k = keys_v[pl.ds(i * 16, 16)]
                return acc + jnp.where(k >= ts, jnp.int32(1), jnp.int32(0))
            acc = lax.fori_loop(0, KV, cbody, jnp.zeros((16,), jnp.int32),
                                unroll=8)
            return lane_sum(acc)

        def bbody(b, upref):
            cand_u = upref | jnp.left_shift(jnp.int32(1), 31 - b)
            cnt = count_ge(cand_u ^ IMIN)
            return jnp.where(cnt >= CAP, cand_u, upref)
        upref = lax.fori_loop(0, 32, bbody, jnp.int32(0))
        thr = upref ^ IMIN

        def sbody(i, off):
            k = keys_v[pl.ds(i * 16, 16)]
            p = probs_v[pl.ds(i * 16, 16)]
            gt = k > thr
            ids = lax.iota(jnp.int32, 16) + i * 16
            plsc.store_compressed(idxl_v.at[pl.ds(off, 16)], ids, mask=gt)
            plsc.store_compressed(gvals_v.at[pl.ds(off, 16)], p, mask=gt)
            return off + popcnt(gt)
        off = lax.fori_loop(0, KV, sbody, jnp.int32(0), unroll=4)

        def tcond(c):
            i, o = c
            return (i < KV) & (o < CAP)

        def tbody(c):
            i, o = c
            k = keys_v[pl.ds(i * 16, 16)]
            p = probs_v[pl.ds(i * 16, 16)]
            eq = k == thr
            ids = lax.iota(jnp.int32, 16) + i * 16
            plsc.store_compressed(idxl_v.at[pl.ds(o, 16)], ids, mask=eq)
            plsc.store_compressed(gvals_v.at[pl.ds(o, 16)], p, mask=eq)
            return i + 1, o + popcnt(eq)
        lax.while_loop(tcond, tbody, (jnp.int32(0), off))

        def zbody(i, c):
            gd_v[pl.ds(i * 16, 16)] = jnp.zeros((16,), jnp.float32)
            return c
        lax.fori_loop(0, KV, zbody, jnp.int32(0), unroll=8)
        for j in range(CAP // 16):
            idxv = idxl_v[pl.ds(j * 16, 16)]
            plsc.store_scatter(gd_v, [idxv], gvals_v[pl.ds(j * 16, 16)])

        pltpu.sync_copy(gd_v, gs_all_sh.at[e])
        plsc.subcore_barrier()
        pltpu.sync_copy(gs_all_sh.at[:, pl.ds(sid * KV, KV)], seg_v)
        for c in range(KV // 16):
            acc = seg_v[0, pl.ds(c * 16, 16)]
            for ee in range(1, E):
                acc = acc + seg_v[ee, pl.ds(c * 16, 16)]
            gsp_v[pl.ds(c * 16, 16)] = acc
        pltpu.sync_copy(gsp_v, gs_tot_sh.at[pl.ds(sid * KV, KV)])
        plsc.subcore_barrier()
        pltpu.sync_copy(gs_tot_sh, gs_v)

        for j in range(CAP // 16):
            idxv = idxl_v[pl.ds(j * 16, 16)]
            gsv = plsc.load_gather(gs_v, [idxv])
            gn_v[pl.ds(j * 16, 16)] = gvals_v[pl.ds(j * 16, 16)] / gsv
        pltpu.sync_copy(idxl_v.at[pl.ds(0, CAP)], tok_hbm.at[e])
        pltpu.sync_copy(gn_v, gn_hbm.at[e])


def _routing(keys_T, probs_T):
    mesh = plsc.VectorSubcoreMesh(**_MESH)
    f = pl.kernel(
        _routing_body,
        out_type=[
            jax.ShapeDtypeStruct((E, CAP), jnp.int32),
            jax.ShapeDtypeStruct((E, CAP), jnp.float32),
        ],
        mesh=mesh,
        compiler_params=pltpu.CompilerParams(needs_layout_passes=False),
        scratch_types=[
            pltpu.VMEM((N,), jnp.int32),
            pltpu.VMEM((N,), jnp.float32),
            pltpu.VMEM((N,), jnp.float32),
            pltpu.VMEM((E, KV), jnp.float32),
            pltpu.VMEM((KV,), jnp.float32),
            pltpu.VMEM((N,), jnp.float32),
            pltpu.VMEM((CAP + 16,), jnp.int32),
            pltpu.VMEM((CAP + 16,), jnp.float32),
            pltpu.VMEM((CAP,), jnp.float32),
            pltpu.VMEM_SHARED((E, N), jnp.float32),
            pltpu.VMEM_SHARED((N,), jnp.float32),
        ],
    )
    return f(keys_T, probs_T)


GCH = 64


D2 = D // 2


def _gather(xn_i32, tok_g, rg):
    rpw = rg // NW

    def body(xn_hbm, idx_hbm, xg_hbm, idx_v, rows_v, sem):
        wid = lax.axis_index("s") * NC + lax.axis_index("c")
        base = wid * rpw
        pltpu.sync_copy(idx_hbm.at[pl.ds(base, rpw)], idx_v)
        for q in range(rpw // GCH):
            pltpu.async_copy(xn_hbm.at[idx_v.at[pl.ds(q * GCH, GCH)]],
                             rows_v, sem).wait()
            pltpu.sync_copy(rows_v, xg_hbm.at[pl.ds(base + q * GCH, GCH)])

    f = pl.kernel(
        body,
        out_type=jax.ShapeDtypeStruct((rg, D2), jnp.int32),
        mesh=plsc.VectorSubcoreMesh(**_MESH),
        compiler_params=pltpu.CompilerParams(needs_layout_passes=False),
        scratch_types=[
            pltpu.VMEM((rpw,), jnp.int32),
            pltpu.VMEM((GCH, D2), jnp.int32),
            pltpu.SemaphoreType.DMA,
        ],
    )
    return f(xn_i32, tok_g)


OSR = N // NS
CBLK = 128


def _scatter(yg_flat, tok3, base, rg):
    asr = rg // NS
    nq = asr // 128

    def body(yg_hbm, idx3_hbm, base_hbm, out_hbm, idx_v, rows_v, acc_sh):
        cid = lax.axis_index("c")
        sid = lax.axis_index("s")
        pltpu.sync_copy(idx3_hbm.at[sid], idx_v)
        for p in range(D // (NC * CBLK)):
            c0 = cid * (D // NC) + p * CBLK
            pltpu.sync_copy(
                base_hbm.at[pl.ds(sid * OSR, OSR), pl.ds(c0, CBLK)],
                acc_sh.at[pl.ds(sid * OSR, OSR)])
            plsc.subcore_barrier()
            pltpu.sync_copy(yg_hbm.at[pl.ds(sid * asr, asr), pl.ds(c0, CBLK)],
                            rows_v)
            for q in range(nq):
                pltpu.sync_copy(rows_v.at[pl.ds(q * 128, 128)],
                                acc_sh.at[idx_v.at[q]], add=True)
            plsc.subcore_barrier()
            pltpu.sync_copy(acc_sh.at[pl.ds(sid * OSR, OSR)],
                            out_hbm.at[pl.ds(sid * OSR, OSR), pl.ds(c0, CBLK)])

    f = pl.kernel(
        body,
        out_type=jax.ShapeDtypeStruct((N, D), jnp.float32),
        mesh=plsc.VectorSubcoreMesh(**_MESH),
        compiler_params=pltpu.CompilerParams(needs_layout_passes=False),
        scratch_types=[
            pltpu.VMEM((nq, 128), jnp.int32),
            pltpu.VMEM((asr, CBLK), jnp.float32),
            pltpu.VMEM_SHARED((N, CBLK), jnp.float32),
        ],
    )
    return f(yg_flat, tok3, base)


EG = E // 2


def kernel(x, norm_weight, router_weight, router_bias, fc1_weight, fc1_bias,
           fc2_weight, fc2_bias):
    x_flat = x.reshape(N, D)
    xn, keys_T, probs_T = _router(x_flat, norm_weight, router_weight,
                                  router_bias)
    token_idx, g_norm = _routing(keys_T, probs_T)
    tok_flat = token_idx.reshape(N)
    xn_i32 = lax.bitcast_convert_type(xn.reshape(N, D2, 2), jnp.int32)
    rg = EG * CAP
    out = jnp.zeros((N, D), jnp.float32)
    for g in range(E // EG):
        tok_g = lax.dynamic_slice(tok_flat, (g * rg,), (rg,))
        xg_g = lax.bitcast_convert_type(_gather(xn_i32, tok_g, rg),
                                        jnp.bfloat16).reshape(rg, D)
        yg_g = _mlp(xg_g, g_norm, fc1_weight, fc1_bias, fc2_weight,
                    fc2_bias, g * EG, EG)
        out = _scatter(yg_g.reshape(rg, D),
                       tok_g.reshape(NS, rg // NS // 128, 128), out, rg)
    return out.reshape(B, L, D)


# speedup vs baseline: 1.5539x; 1.5539x over previous
import functools
import math

import jax
import jax.numpy as jnp
from jax import lax
from jax.experimental import pallas as pl
from jax.experimental.pallas import tpu as pltpu
from jax.experimental.pallas import tpu_sc as plsc

B, L, D, E, FF = 2, 2048, 1024, 16, 4096
N = B * L
CAP = N // E
EPS = 1e-05

TOK_BLK = 512
FF_BLK = 1024


def _router_body(x_ref, nw_ref, rw_ref, rb_ref, xn_ref, kt_ref, pt_ref):
    x = x_ref[...]
    var = jnp.mean(x * x, axis=1, keepdims=True)
    xn = x * lax.rsqrt(var + EPS) * nw_ref[...]
    xn_ref[...] = xn
    lt = lax.dot_general(rw_ref[...], xn, (((1,), (1,)), ((), ())),
                         preferred_element_type=jnp.float32)
    lt = lt + rb_ref[...].reshape(E, 1)
    kb = lax.bitcast_convert_type(lt, jnp.int32)
    kt_ref[...] = jnp.where(kb < 0, kb ^ jnp.int32(0x7FFFFFFF), kb)
    m = jnp.max(lt, axis=0, keepdims=True)
    ex = jnp.exp(lt - m)
    pt_ref[...] = ex / jnp.sum(ex, axis=0, keepdims=True)


def _router(x_flat, norm_weight, router_weight, router_bias):
    grid = (N // TOK_BLK,)
    return pl.pallas_call(
        _router_body,
        grid=grid,
        in_specs=[
            pl.BlockSpec((TOK_BLK, D), lambda i: (i, 0)),
            pl.BlockSpec((1, D), lambda i: (0, 0)),
            pl.BlockSpec((E, D), lambda i: (0, 0)),
            pl.BlockSpec((1, E), lambda i: (0, 0)),
        ],
        out_specs=[
            pl.BlockSpec((TOK_BLK, D), lambda i: (i, 0)),
            pl.BlockSpec((E, TOK_BLK), lambda i: (0, i)),
            pl.BlockSpec((E, TOK_BLK), lambda i: (0, i)),
        ],
        out_shape=[
            jax.ShapeDtypeStruct((N, D), jnp.float32),
            jax.ShapeDtypeStruct((E, N), jnp.int32),
            jax.ShapeDtypeStruct((E, N), jnp.float32),
        ],
    )(x_flat, norm_weight.reshape(1, D), router_weight, router_bias.reshape(1, E))


def _gelu_exact(h):
    return 0.5 * h * (1.0 + lax.erf(h * (1.0 / math.sqrt(2.0))))


def _mlp_body(xg_ref, g_ref, w1_ref, b1_ref, w2_ref, b2_ref, out_ref, acc_ref):
    f = pl.program_id(1)
    xg = xg_ref[0].astype(jnp.bfloat16)
    w1 = w1_ref[0].astype(jnp.bfloat16)
    h = lax.dot_general(xg, w1, (((1,), (1,)), ((), ())),
                        preferred_element_type=jnp.float32)
    h = h + b1_ref[0]
    h = _gelu_exact(h)
    w2 = w2_ref[0].astype(jnp.bfloat16)
    y = lax.dot_general(h.astype(jnp.bfloat16), w2, (((1,), (1,)), ((), ())),
                        preferred_element_type=jnp.float32)

    @pl.when(f == 0)
    def _():
        acc_ref[...] = y + b2_ref[0]

    @pl.when(f > 0)
    def _():
        acc_ref[...] += y

    @pl.when(f == FF // FF_BLK - 1)
    def _():
        out_ref[0] = acc_ref[...] * g_ref[0].reshape(CAP, 1)


def _mlp(xg_g, g_norm, fc1_weight, fc1_bias, fc2_weight, fc2_bias, e0, eg):
    grid = (eg, FF // FF_BLK)
    return pl.pallas_call(
        _mlp_body,
        grid=grid,
        in_specs=[
            pl.BlockSpec((1, CAP, D), lambda e, f: (e, 0, 0)),
            pl.BlockSpec((1, 1, CAP), lambda e, f: (e + e0, 0, 0)),
            pl.BlockSpec((1, FF_BLK, D), lambda e, f: (e + e0, f, 0)),
            pl.BlockSpec((1, 1, FF_BLK), lambda e, f: (e + e0, 0, f)),
            pl.BlockSpec((1, D, FF_BLK), lambda e, f: (e + e0, 0, f)),
            pl.BlockSpec((1, 1, D), lambda e, f: (e + e0, 0, 0)),
        ],
        out_specs=pl.BlockSpec((1, CAP, D), lambda e, f: (e, 0, 0)),
        out_shape=jax.ShapeDtypeStruct((eg, CAP, D), jnp.float32),
        scratch_shapes=[pltpu.VMEM((CAP, D), jnp.float32)],
        compiler_params=pltpu.CompilerParams(
            dimension_semantics=("parallel", "arbitrary")),
    )(xg_g.reshape(eg, CAP, D), g_norm.reshape(E, 1, CAP),
      fc1_weight, fc1_bias.reshape(E, 1, FF),
      fc2_weight, fc2_bias.reshape(E, 1, D))


NC, NS = 2, 16
NW = NC * NS
KV = N // 16
_MESH = dict(core_axis_name="c", subcore_axis_name="s")
IMIN = -2147483648


def _routing_body(kt_hbm, pt_hbm, tok_hbm, gn_hbm,
                  keys_v, probs_v, gd_v, seg_v, gsp_v, gs_v,
                  idxl_v, gvals_v, gn_v, gs_all_sh, gs_tot_sh):
    cid = lax.axis_index("c")
    sid = lax.axis_index("s")

    @pl.when(cid == 0)
    def _():
        e = sid
        pltpu.sync_copy(kt_hbm.at[e], keys_v)
        pltpu.sync_copy(pt_hbm.at[e], probs_v)

        def lane_sum(v):
            s = v[0]
            for l in range(1, 16):
                s = s + v[l]
            return s

        def popcnt(m):
            return lane_sum(jnp.where(m, jnp.int32(1), jnp.int32(0)))

        def count_ge(ts):
            def cbody(i, acc):
                k = keys_v[pl.ds(i * 16, 16)]
                return acc + jnp.where(k >= ts, jnp.int32(1), jnp.int32(0))
            acc = lax.fori_loop(0, KV, cbody, jnp.zeros((16,), jnp.int32),
                                unroll=8)
            return lane_sum(acc)

        def bbody(b, upref):
            cand_u = upref | jnp.left_shift(jnp.int32(1), 31 - b)
            cnt = count_ge(cand_u ^ IMIN)
            return jnp.where(cnt >= CAP, cand_u, upref)
        upref = lax.fori_loop(0, 32, bbody, jnp.int32(0))
        thr = upref ^ IMIN

        def sbody(i, off):
            k = keys_v[pl.ds(i * 16, 16)]
            p = probs_v[pl.ds(i * 16, 16)]
            gt = k > thr
            ids = lax.iota(jnp.int32, 16) + i * 16
            plsc.store_compressed(idxl_v.at[pl.ds(off, 16)], ids, mask=gt)
            plsc.store_compressed(gvals_v.at[pl.ds(off, 16)], p, mask=gt)
            return off + popcnt(gt)
        off = lax.fori_loop(0, KV, sbody, jnp.int32(0), unroll=4)

        def tcond(c):
            i, o = c
            return (i < KV) & (o < CAP)

        def tbody(c):
            i, o = c
            k = keys_v[pl.ds(i * 16, 16)]
            p = probs_v[pl.ds(i * 16, 16)]
            eq = k == thr
            ids = lax.iota(jnp.int32, 16) + i * 16
            plsc.store_compressed(idxl_v.at[pl.ds(o, 16)], ids, mask=eq)
            plsc.store_compressed(gvals_v.at[pl.ds(o, 16)], p, mask=eq)
            return i + 1, o + popcnt(eq)
        lax.while_loop(tcond, tbody, (jnp.int32(0), off))

        def zbody(i, c):
            gd_v[pl.ds(i * 16, 16)] = jnp.zeros((16,), jnp.float32)
            return c
        lax.fori_loop(0, KV, zbody, jnp.int32(0), unroll=8)
        for j in range(CAP // 16):
            idxv = idxl_v[pl.ds(j * 16, 16)]
            plsc.store_scatter(gd_v, [idxv], gvals_v[pl.ds(j * 16, 16)])

        pltpu.sync_copy(gd_v, gs_all_sh.at[e])
        plsc.subcore_barrier()
        pltpu.sync_copy(gs_all_sh.at[:, pl.ds(sid * KV, KV)], seg_v)
        for c in range(KV // 16):
            acc = seg_v[0, pl.ds(c * 16, 16)]
            for ee in range(1, E):
                acc = acc + seg_v[ee, pl.ds(c * 16, 16)]
            gsp_v[pl.ds(c * 16, 16)] = acc
        pltpu.sync_copy(gsp_v, gs_tot_sh.at[pl.ds(sid * KV, KV)])
        plsc.subcore_barrier()
        pltpu.sync_copy(gs_tot_sh, gs_v)

        for j in range(CAP // 16):
            idxv = idxl_v[pl.ds(j * 16, 16)]
            gsv = plsc.load_gather(gs_v, [idxv])
            gn_v[pl.ds(j * 16, 16)] = gvals_v[pl.ds(j * 16, 16)] / gsv
        pltpu.sync_copy(idxl_v.at[pl.ds(0, CAP)], tok_hbm.at[e])
        pltpu.sync_copy(gn_v, gn_hbm.at[e])


def _routing(keys_T, probs_T):
    mesh = plsc.VectorSubcoreMesh(**_MESH)
    f = pl.kernel(
        _routing_body,
        out_type=[
            jax.ShapeDtypeStruct((E, CAP), jnp.int32),
            jax.ShapeDtypeStruct((E, CAP), jnp.float32),
        ],
        mesh=mesh,
        compiler_params=pltpu.CompilerParams(needs_layout_passes=False),
        scratch_types=[
            pltpu.VMEM((N,), jnp.int32),
            pltpu.VMEM((N,), jnp.float32),
            pltpu.VMEM((N,), jnp.float32),
            pltpu.VMEM((E, KV), jnp.float32),
            pltpu.VMEM((KV,), jnp.float32),
            pltpu.VMEM((N,), jnp.float32),
            pltpu.VMEM((CAP + 16,), jnp.int32),
            pltpu.VMEM((CAP + 16,), jnp.float32),
            pltpu.VMEM((CAP,), jnp.float32),
            pltpu.VMEM_SHARED((E, N), jnp.float32),
            pltpu.VMEM_SHARED((N,), jnp.float32),
        ],
    )
    return f(keys_T, probs_T)


GCH = 64


def _gather(xn, tok_g, rg):
    rpw = rg // NW

    def body(xn_hbm, idx_hbm, xg_hbm, idx_v, rows_v, sem):
        wid = lax.axis_index("s") * NC + lax.axis_index("c")
        base = wid * rpw
        pltpu.sync_copy(idx_hbm.at[pl.ds(base, rpw)], idx_v)
        for q in range(rpw // GCH):
            pltpu.async_copy(xn_hbm.at[idx_v.at[pl.ds(q * GCH, GCH)]],
                             rows_v, sem).wait()
            pltpu.sync_copy(rows_v, xg_hbm.at[pl.ds(base + q * GCH, GCH)])

    f = pl.kernel(
        body,
        out_type=jax.ShapeDtypeStruct((rg, D), jnp.float32),
        mesh=plsc.VectorSubcoreMesh(**_MESH),
        compiler_params=pltpu.CompilerParams(needs_layout_passes=False),
        scratch_types=[
            pltpu.VMEM((rpw,), jnp.int32),
            pltpu.VMEM((GCH, D), jnp.float32),
            pltpu.SemaphoreType.DMA,
        ],
    )
    return f(xn, tok_g)


OSR = N // NS
CBLK = 128


def _scatter(yg_flat, tok3, base, rg):
    asr = rg // NS
    nq = asr // 128

    def body(yg_hbm, idx3_hbm, base_hbm, out_hbm, idx_v, rows_v, acc_sh):
        cid = lax.axis_index("c")
        sid = lax.axis_index("s")
        pltpu.sync_copy(idx3_hbm.at[sid], idx_v)
        for p in range(D // (NC * CBLK)):
            c0 = cid * (D // NC) + p * CBLK
            pltpu.sync_copy(
                base_hbm.at[pl.ds(sid * OSR, OSR), pl.ds(c0, CBLK)],
                acc_sh.at[pl.ds(sid * OSR, OSR)])
            plsc.subcore_barrier()
            pltpu.sync_copy(yg_hbm.at[pl.ds(sid * asr, asr), pl.ds(c0, CBLK)],
                            rows_v)
            for q in range(nq):
                pltpu.sync_copy(rows_v.at[pl.ds(q * 128, 128)],
                                acc_sh.at[idx_v.at[q]], add=True)
            plsc.subcore_barrier()
            pltpu.sync_copy(acc_sh.at[pl.ds(sid * OSR, OSR)],
                            out_hbm.at[pl.ds(sid * OSR, OSR), pl.ds(c0, CBLK)])

    f = pl.kernel(
        body,
        out_type=jax.ShapeDtypeStruct((N, D), jnp.float32),
        mesh=plsc.VectorSubcoreMesh(**_MESH),
        compiler_params=pltpu.CompilerParams(needs_layout_passes=False),
        scratch_types=[
            pltpu.VMEM((nq, 128), jnp.int32),
            pltpu.VMEM((asr, CBLK), jnp.float32),
            pltpu.VMEM_SHARED((N, CBLK), jnp.float32),
        ],
    )
    return f(yg_flat, tok3, base)


def kernel(x, norm_weight, router_weight, router_bias, fc1_weight, fc1_bias,
           fc2_weight, fc2_bias):
    x_flat = x.reshape(N, D)
    xn, keys_T, probs_T = _router(x_flat, norm_weight, router_weight,
                                  router_bias)
    token_idx, g_norm = _routing(keys_T, probs_T)
    tok_flat = token_idx.reshape(N)
    xg = _gather(xn, tok_flat, N)
    yg = _mlp(xg, g_norm, fc1_weight, fc1_bias, fc2_weight, fc2_bias, 0, E)
    out = _scatter(yg.reshape(N, D), token_idx.reshape(NS, N // NS // 128, 128),
                   jnp.zeros((N, D), jnp.float32), N)
    return out.reshape(B, L, D)


# speedup vs baseline: 1.6073x; 1.0344x over previous
import functools
import math

import jax
import jax.numpy as jnp
from jax import lax
from jax.experimental import pallas as pl
from jax.experimental.pallas import tpu as pltpu
from jax.experimental.pallas import tpu_sc as plsc

B, L, D, E, FF = 2, 2048, 1024, 16, 4096
N = B * L
CAP = N // E
EPS = 1e-05

TOK_BLK = 512
FF_BLK = 2048


def _router_body(x_ref, nw_ref, rw_ref, rb_ref, xn_ref, kt_ref, pt_ref):
    x = x_ref[...]
    var = jnp.mean(x * x, axis=1, keepdims=True)
    xn = x * lax.rsqrt(var + EPS) * nw_ref[...]
    xn_ref[...] = xn
    lt = lax.dot_general(rw_ref[...], xn, (((1,), (1,)), ((), ())),
                         preferred_element_type=jnp.float32)
    lt = lt + rb_ref[...].reshape(E, 1)
    kb = lax.bitcast_convert_type(lt, jnp.int32)
    kt_ref[...] = jnp.where(kb < 0, kb ^ jnp.int32(0x7FFFFFFF), kb)
    m = jnp.max(lt, axis=0, keepdims=True)
    ex = jnp.exp(lt - m)
    pt_ref[...] = ex / jnp.sum(ex, axis=0, keepdims=True)


def _router(x_flat, norm_weight, router_weight, router_bias):
    grid = (N // TOK_BLK,)
    return pl.pallas_call(
        _router_body,
        grid=grid,
        in_specs=[
            pl.BlockSpec((TOK_BLK, D), lambda i: (i, 0)),
            pl.BlockSpec((1, D), lambda i: (0, 0)),
            pl.BlockSpec((E, D), lambda i: (0, 0)),
            pl.BlockSpec((1, E), lambda i: (0, 0)),
        ],
        out_specs=[
            pl.BlockSpec((TOK_BLK, D), lambda i: (i, 0)),
            pl.BlockSpec((E, TOK_BLK), lambda i: (0, i)),
            pl.BlockSpec((E, TOK_BLK), lambda i: (0, i)),
        ],
        out_shape=[
            jax.ShapeDtypeStruct((N, D), jnp.float32),
            jax.ShapeDtypeStruct((E, N), jnp.int32),
            jax.ShapeDtypeStruct((E, N), jnp.float32),
        ],
    )(x_flat, norm_weight.reshape(1, D), router_weight, router_bias.reshape(1, E))


def _gelu_exact(h):
    return 0.5 * h * (1.0 + lax.erf(h * (1.0 / math.sqrt(2.0))))


def _mlp_body(xg_ref, g_ref, w1_ref, b1_ref, w2_ref, b2_ref, out_ref, acc_ref):
    f = pl.program_id(1)
    xg = xg_ref[0].astype(jnp.bfloat16)
    w1 = w1_ref[0].astype(jnp.bfloat16)
    h = lax.dot_general(xg, w1, (((1,), (1,)), ((), ())),
                        preferred_element_type=jnp.float32)
    h = h + b1_ref[0]
    h = _gelu_exact(h)
    w2 = w2_ref[0].astype(jnp.bfloat16)
    y = lax.dot_general(h.astype(jnp.bfloat16), w2, (((1,), (1,)), ((), ())),
                        preferred_element_type=jnp.float32)

    @pl.when(f == 0)
    def _():
        acc_ref[...] = y + b2_ref[0]

    @pl.when(f > 0)
    def _():
        acc_ref[...] += y

    @pl.when(f == FF // FF_BLK - 1)
    def _():
        out_ref[0] = acc_ref[...] * g_ref[0].reshape(CAP, 1)


def _mlp(xg_g, g_norm, fc1_weight, fc1_bias, fc2_weight, fc2_bias, e0, eg):
    grid = (eg, FF // FF_BLK)
    return pl.pallas_call(
        _mlp_body,
        grid=grid,
        in_specs=[
            pl.BlockSpec((1, CAP, D), lambda e, f: (e, 0, 0)),
            pl.BlockSpec((1, 1, CAP), lambda e, f: (e + e0, 0, 0)),
            pl.BlockSpec((1, FF_BLK, D), lambda e, f: (e + e0, f, 0)),
            pl.BlockSpec((1, 1, FF_BLK), lambda e, f: (e + e0, 0, f)),
            pl.BlockSpec((1, D, FF_BLK), lambda e, f: (e + e0, 0, f)),
            pl.BlockSpec((1, 1, D), lambda e, f: (e + e0, 0, 0)),
        ],
        out_specs=pl.BlockSpec((1, CAP, D), lambda e, f: (e, 0, 0)),
        out_shape=jax.ShapeDtypeStruct((eg, CAP, D), jnp.float32),
        scratch_shapes=[pltpu.VMEM((CAP, D), jnp.float32)],
        compiler_params=pltpu.CompilerParams(
            dimension_semantics=("parallel", "arbitrary")),
    )(xg_g.reshape(eg, CAP, D), g_norm.reshape(E, 1, CAP),
      fc1_weight, fc1_bias.reshape(E, 1, FF),
      fc2_weight, fc2_bias.reshape(E, 1, D))


NC, NS = 2, 16
NW = NC * NS
KV = N // 16
_MESH = dict(core_axis_name="c", subcore_axis_name="s")
IMIN = -2147483648


def _routing_body(kt_hbm, pt_hbm, tok_hbm, gn_hbm,
                  keys_v, probs_v, gd_v, seg_v, gsp_v, gs_v,
                  idxl_v, gvals_v, gn_v, gs_all_sh, gs_tot_sh):
    cid = lax.axis_index("c")
    sid = lax.axis_index("s")

    @pl.when(cid == 0)
    def _():
        e = sid
        pltpu.sync_copy(kt_hbm.at[e], keys_v)
        pltpu.sync_copy(pt_hbm.at[e], probs_v)

        def lane_sum(v):
            s = v[0]
            for l in range(1, 16):
                s = s + v[l]
            return s

        def popcnt(m):
            return lane_sum(jnp.where(m, jnp.int32(1), jnp.int32(0)))

        def count_ge(ts):
            def cbody(i, acc):
                k = keys_v[pl.ds(i * 16, 16)]
                return acc + jnp.where(k >= ts, jnp.int32(1), jnp.int32(0))
            acc = lax.fori_loop(0, KV, cbody, jnp.zeros((16,), jnp.int32),
                                unroll=8)
            return lane_sum(acc)

        def bbody(b, upref):
            cand_u = upref | jnp.left_shift(jnp.int32(1), 31 - b)
            cnt = count_ge(cand_u ^ IMIN)
            return jnp.where(cnt >= CAP, cand_u, upref)
        upref = lax.fori_loop(0, 32, bbody, jnp.int32(0))
        thr = upref ^ IMIN

        def sbody(i, off):
            k = keys_v[pl.ds(i * 16, 16)]
            p = probs_v[pl.ds(i * 16, 16)]
            gt = k > thr
            ids = lax.iota(jnp.int32, 16) + i * 16
            plsc.store_compressed(idxl_v.at[pl.ds(off, 16)], ids, mask=gt)
            plsc.store_compressed(gvals_v.at[pl.ds(off, 16)], p, mask=gt)
            return off + popcnt(gt)
        off = lax.fori_loop(0, KV, sbody, jnp.int32(0), unroll=4)

        def tcond(c):
            i, o = c
            return (i < KV) & (o < CAP)

        def tbody(c):
            i, o = c
            k = keys_v[pl.ds(i * 16, 16)]
            p = probs_v[pl.ds(i * 16, 16)]
            eq = k == thr
            ids = lax.iota(jnp.int32, 16) + i * 16
            plsc.store_compressed(idxl_v.at[pl.ds(o, 16)], ids, mask=eq)
            plsc.store_compressed(gvals_v.at[pl.ds(o, 16)], p, mask=eq)
            return i + 1, o + popcnt(eq)
        lax.while_loop(tcond, tbody, (jnp.int32(0), off))

        def zbody(i, c):
            gd_v[pl.ds(i * 16, 16)] = jnp.zeros((16,), jnp.float32)
            return c
        lax.fori_loop(0, KV, zbody, jnp.int32(0), unroll=8)
        for j in range(CAP // 16):
            idxv = idxl_v[pl.ds(j * 16, 16)]
            plsc.store_scatter(gd_v, [idxv], gvals_v[pl.ds(j * 16, 16)])

        pltpu.sync_copy(gd_v, gs_all_sh.at[e])
        plsc.subcore_barrier()
        pltpu.sync_copy(gs_all_sh.at[:, pl.ds(sid * KV, KV)], seg_v)
        for c in range(KV // 16):
            acc = seg_v[0, pl.ds(c * 16, 16)]
            for ee in range(1, E):
                acc = acc + seg_v[ee, pl.ds(c * 16, 16)]
            gsp_v[pl.ds(c * 16, 16)] = acc
        pltpu.sync_copy(gsp_v, gs_tot_sh.at[pl.ds(sid * KV, KV)])
        plsc.subcore_barrier()
        pltpu.sync_copy(gs_tot_sh, gs_v)

        for j in range(CAP // 16):
            idxv = idxl_v[pl.ds(j * 16, 16)]
            gsv = plsc.load_gather(gs_v, [idxv])
            gn_v[pl.ds(j * 16, 16)] = gvals_v[pl.ds(j * 16, 16)] / gsv
        pltpu.sync_copy(idxl_v.at[pl.ds(0, CAP)], tok_hbm.at[e])
        pltpu.sync_copy(gn_v, gn_hbm.at[e])


def _routing(keys_T, probs_T):
    mesh = plsc.VectorSubcoreMesh(**_MESH)
    f = pl.kernel(
        _routing_body,
        out_type=[
            jax.ShapeDtypeStruct((E, CAP), jnp.int32),
            jax.ShapeDtypeStruct((E, CAP), jnp.float32),
        ],
        mesh=mesh,
        compiler_params=pltpu.CompilerParams(needs_layout_passes=False),
        scratch_types=[
            pltpu.VMEM((N,), jnp.int32),
            pltpu.VMEM((N,), jnp.float32),
            pltpu.VMEM((N,), jnp.float32),
            pltpu.VMEM((E, KV), jnp.float32),
            pltpu.VMEM((KV,), jnp.float32),
            pltpu.VMEM((N,), jnp.float32),
            pltpu.VMEM((CAP + 16,), jnp.int32),
            pltpu.VMEM((CAP + 16,), jnp.float32),
            pltpu.VMEM((CAP,), jnp.float32),
            pltpu.VMEM_SHARED((E, N), jnp.float32),
            pltpu.VMEM_SHARED((N,), jnp.float32),
        ],
    )
    return f(keys_T, probs_T)


GCH = 64


def _gather(xn, tok_g, rg):
    rpw = rg // NW

    def body(xn_hbm, idx_hbm, xg_hbm, idx_v, rows_v, sem):
        wid = lax.axis_index("s") * NC + lax.axis_index("c")
        base = wid * rpw
        pltpu.sync_copy(idx_hbm.at[pl.ds(base, rpw)], idx_v)
        for q in range(rpw // GCH):
            pltpu.async_copy(xn_hbm.at[idx_v.at[pl.ds(q * GCH, GCH)]],
                             rows_v, sem).wait()
            pltpu.sync_copy(rows_v, xg_hbm.at[pl.ds(base + q * GCH, GCH)])

    f = pl.kernel(
        body,
        out_type=jax.ShapeDtypeStruct((rg, D), jnp.float32),
        mesh=plsc.VectorSubcoreMesh(**_MESH),
        compiler_params=pltpu.CompilerParams(needs_layout_passes=False),
        scratch_types=[
            pltpu.VMEM((rpw,), jnp.int32),
            pltpu.VMEM((GCH, D), jnp.float32),
            pltpu.SemaphoreType.DMA,
        ],
    )
    return f(xn, tok_g)


OSR = N // NS
CBLK = 128


def _scatter(yg_flat, tok3, base, rg):
    asr = rg // NS
    nq = asr // 128

    def body(yg_hbm, idx3_hbm, base_hbm, out_hbm, idx_v, rows_v, acc_sh):
        cid = lax.axis_index("c")
        sid = lax.axis_index("s")
        pltpu.sync_copy(idx3_hbm.at[sid], idx_v)
        for p in range(D // (NC * CBLK)):
            c0 = cid * (D // NC) + p * CBLK
            pltpu.sync_copy(
                base_hbm.at[pl.ds(sid * OSR, OSR), pl.ds(c0, CBLK)],
                acc_sh.at[pl.ds(sid * OSR, OSR)])
            plsc.subcore_barrier()
            pltpu.sync_copy(yg_hbm.at[pl.ds(sid * asr, asr), pl.ds(c0, CBLK)],
                            rows_v)
            for q in range(nq):
                pltpu.sync_copy(rows_v.at[pl.ds(q * 128, 128)],
                                acc_sh.at[idx_v.at[q]], add=True)
            plsc.subcore_barrier()
            pltpu.sync_copy(acc_sh.at[pl.ds(sid * OSR, OSR)],
                            out_hbm.at[pl.ds(sid * OSR, OSR), pl.ds(c0, CBLK)])

    f = pl.kernel(
        body,
        out_type=jax.ShapeDtypeStruct((N, D), jnp.float32),
        mesh=plsc.VectorSubcoreMesh(**_MESH),
        compiler_params=pltpu.CompilerParams(needs_layout_passes=False),
        scratch_types=[
            pltpu.VMEM((nq, 128), jnp.int32),
            pltpu.VMEM((asr, CBLK), jnp.float32),
            pltpu.VMEM_SHARED((N, CBLK), jnp.float32),
        ],
    )
    return f(yg_flat, tok3, base)


def kernel(x, norm_weight, router_weight, router_bias, fc1_weight, fc1_bias,
           fc2_weight, fc2_bias):
    x_flat = x.reshape(N, D)
    xn, keys_T, probs_T = _router(x_flat, norm_weight, router_weight,
                                  router_bias)
    token_idx, g_norm = _routing(keys_T, probs_T)
    tok_flat = token_idx.reshape(N)
    xg = _gather(xn, tok_flat, N)
    yg = _mlp(xg, g_norm, fc1_weight, fc1_bias, fc2_weight, fc2_bias, 0, E)
    out = _scatter(yg.reshape(N, D), token_idx.reshape(NS, N // NS // 128, 128),
                   jnp.zeros((N, D), jnp.float32), N)
    return out.reshape(B, L, D)


# speedup vs baseline: 1.6639x; 1.0352x over previous
import functools
import math

import jax
import jax.numpy as jnp
from jax import lax
from jax.experimental import pallas as pl
from jax.experimental.pallas import tpu as pltpu
from jax.experimental.pallas import tpu_sc as plsc

B, L, D, E, FF = 2, 2048, 1024, 16, 4096
N = B * L
CAP = N // E
D2 = D // 2
EPS = 1e-05

TOK_BLK = 512
FF_BLK = 2048


def _router_body(x_ref, nw_ref, rw_ref, rb_ref, xn_ref, kt_ref, pt_ref):
    x = x_ref[...]
    var = jnp.mean(x * x, axis=1, keepdims=True)
    xn = x * lax.rsqrt(var + EPS) * nw_ref[...]
    xnb = xn.astype(jnp.bfloat16)
    lo = lax.bitcast_convert_type(xnb[:, :D2], jnp.int16).astype(jnp.int32)
    hi = lax.bitcast_convert_type(xnb[:, D2:], jnp.int16).astype(jnp.int32)
    xn_ref[...] = jnp.left_shift(hi, 16) | (lo & 0xFFFF)
    lt = lax.dot_general(rw_ref[...], xn, (((1,), (1,)), ((), ())),
                         preferred_element_type=jnp.float32)
    lt = lt + rb_ref[...].reshape(E, 1)
    kb = lax.bitcast_convert_type(lt, jnp.int32)
    kt_ref[...] = jnp.where(kb < 0, kb ^ jnp.int32(0x7FFFFFFF), kb)
    m = jnp.max(lt, axis=0, keepdims=True)
    ex = jnp.exp(lt - m)
    pt_ref[...] = ex / jnp.sum(ex, axis=0, keepdims=True)


def _router(x_flat, norm_weight, router_weight, router_bias):
    grid = (N // TOK_BLK,)
    return pl.pallas_call(
        _router_body,
        grid=grid,
        in_specs=[
            pl.BlockSpec((TOK_BLK, D), lambda i: (i, 0)),
            pl.BlockSpec((1, D), lambda i: (0, 0)),
            pl.BlockSpec((E, D), lambda i: (0, 0)),
            pl.BlockSpec((1, E), lambda i: (0, 0)),
        ],
        out_specs=[
            pl.BlockSpec((TOK_BLK, D2), lambda i: (i, 0)),
            pl.BlockSpec((E, TOK_BLK), lambda i: (0, i)),
            pl.BlockSpec((E, TOK_BLK), lambda i: (0, i)),
        ],
        out_shape=[
            jax.ShapeDtypeStruct((N, D2), jnp.int32),
            jax.ShapeDtypeStruct((E, N), jnp.int32),
            jax.ShapeDtypeStruct((E, N), jnp.float32),
        ],
    )(x_flat, norm_weight.reshape(1, D), router_weight, router_bias.reshape(1, E))


def _gelu_exact(h):
    return 0.5 * h * (1.0 + lax.erf(h * (1.0 / math.sqrt(2.0))))


def _mlp_body(xg_ref, g_ref, w1_ref, b1_ref, w2_ref, b2_ref, out_ref, acc_ref):
    f = pl.program_id(1)
    xgi = xg_ref[0]
    xlo = lax.bitcast_convert_type(xgi.astype(jnp.int16), jnp.bfloat16)
    xhi = lax.bitcast_convert_type(
        jnp.right_shift(xgi, 16).astype(jnp.int16), jnp.bfloat16)
    xg = jnp.concatenate([xlo, xhi], axis=1)
    w1 = w1_ref[0].astype(jnp.bfloat16)
    h = lax.dot_general(xg, w1, (((1,), (1,)), ((), ())),
                        preferred_element_type=jnp.float32)
    h = h + b1_ref[0]
    h = _gelu_exact(h)
    w2 = w2_ref[0].astype(jnp.bfloat16)
    y = lax.dot_general(h.astype(jnp.bfloat16), w2, (((1,), (1,)), ((), ())),
                        preferred_element_type=jnp.float32)

    @pl.when(f == 0)
    def _():
        acc_ref[...] = y + b2_ref[0]

    @pl.when(f > 0)
    def _():
        acc_ref[...] += y

    @pl.when(f == FF // FF_BLK - 1)
    def _():
        out_ref[0] = acc_ref[...] * g_ref[0].reshape(CAP, 1)


def _mlp(xg_g, g_norm, fc1_weight, fc1_bias, fc2_weight, fc2_bias, e0, eg):
    grid = (eg, FF // FF_BLK)
    return pl.pallas_call(
        _mlp_body,
        grid=grid,
        in_specs=[
            pl.BlockSpec((1, CAP, D2), lambda e, f: (e, 0, 0)),
            pl.BlockSpec((1, 1, CAP), lambda e, f: (e + e0, 0, 0)),
            pl.BlockSpec((1, FF_BLK, D), lambda e, f: (e + e0, f, 0)),
            pl.BlockSpec((1, 1, FF_BLK), lambda e, f: (e + e0, 0, f)),
            pl.BlockSpec((1, D, FF_BLK), lambda e, f: (e + e0, 0, f)),
            pl.BlockSpec((1, 1, D), lambda e, f: (e + e0, 0, 0)),
        ],
        out_specs=pl.BlockSpec((1, CAP, D), lambda e, f: (e, 0, 0)),
        out_shape=jax.ShapeDtypeStruct((eg, CAP, D), jnp.float32),
        scratch_shapes=[pltpu.VMEM((CAP, D), jnp.float32)],
        compiler_params=pltpu.CompilerParams(
            dimension_semantics=("parallel", "arbitrary")),
    )(xg_g.reshape(eg, CAP, D2), g_norm.reshape(E, 1, CAP),
      fc1_weight, fc1_bias.reshape(E, 1, FF),
      fc2_weight, fc2_bias.reshape(E, 1, D))


NC, NS = 2, 16
NW = NC * NS
KV = N // 16
_MESH = dict(core_axis_name="c", subcore_axis_name="s")
IMIN = -2147483648


def _routing_body(kt_hbm, pt_hbm, tok_hbm, gn_hbm,
                  keys_v, probs_v, gd_v, seg_v, gsp_v, gs_v,
                  idxl_v, gvals_v, gn_v, gs_all_sh, gs_tot_sh):
    cid = lax.axis_index("c")
    sid = lax.axis_index("s")

    @pl.when(cid == 0)
    def _():
        e = sid
        pltpu.sync_copy(kt_hbm.at[e], keys_v)
        pltpu.sync_copy(pt_hbm.at[e], probs_v)

        def lane_sum(v):
            s = v[0]
            for l in range(1, 16):
                s = s + v[l]
            return s

        def popcnt(m):
            return lane_sum(jnp.where(m, jnp.int32(1), jnp.int32(0)))

        def count_ge(ts):
            def cbody(i, acc):
                k = keys_v[pl.ds(i * 16, 16)]
                return acc + jnp.where(k >= ts, jnp.int32(1), jnp.int32(0))
            acc = lax.fori_loop(0, KV, cbody, jnp.zeros((16,), jnp.int32),
                                unroll=8)
            return lane_sum(acc)

        def bbody(b, upref):
            cand_u = upref | jnp.left_shift(jnp.int32(1), 31 - b)
            cnt = count_ge(cand_u ^ IMIN)
            return jnp.where(cnt >= CAP, cand_u, upref)
        upref = lax.fori_loop(0, 32, bbody, jnp.int32(0))
        thr = upref ^ IMIN

        def sbody(i, off):
            k = keys_v[pl.ds(i * 16, 16)]
            p = probs_v[pl.ds(i * 16, 16)]
            gt = k > thr
            ids = lax.iota(jnp.int32, 16) + i * 16
            plsc.store_compressed(idxl_v.at[pl.ds(off, 16)], ids, mask=gt)
            plsc.store_compressed(gvals_v.at[pl.ds(off, 16)], p, mask=gt)
            return off + popcnt(gt)
        off = lax.fori_loop(0, KV, sbody, jnp.int32(0), unroll=4)

        def tcond(c):
            i, o = c
            return (i < KV) & (o < CAP)

        def tbody(c):
            i, o = c
            k = keys_v[pl.ds(i * 16, 16)]
            p = probs_v[pl.ds(i * 16, 16)]
            eq = k == thr
            ids = lax.iota(jnp.int32, 16) + i * 16
            plsc.store_compressed(idxl_v.at[pl.ds(o, 16)], ids, mask=eq)
            plsc.store_compressed(gvals_v.at[pl.ds(o, 16)], p, mask=eq)
            return i + 1, o + popcnt(eq)
        lax.while_loop(tcond, tbody, (jnp.int32(0), off))

        def zbody(i, c):
            gd_v[pl.ds(i * 16, 16)] = jnp.zeros((16,), jnp.float32)
            return c
        lax.fori_loop(0, KV, zbody, jnp.int32(0), unroll=8)
        for j in range(CAP // 16):
            idxv = idxl_v[pl.ds(j * 16, 16)]
            plsc.store_scatter(gd_v, [idxv], gvals_v[pl.ds(j * 16, 16)])

        pltpu.sync_copy(gd_v, gs_all_sh.at[e])
        plsc.subcore_barrier()
        pltpu.sync_copy(gs_all_sh.at[:, pl.ds(sid * KV, KV)], seg_v)
        for c in range(KV // 16):
            acc = seg_v[0, pl.ds(c * 16, 16)]
            for ee in range(1, E):
                acc = acc + seg_v[ee, pl.ds(c * 16, 16)]
            gsp_v[pl.ds(c * 16, 16)] = acc
        pltpu.sync_copy(gsp_v, gs_tot_sh.at[pl.ds(sid * KV, KV)])
        plsc.subcore_barrier()
        pltpu.sync_copy(gs_tot_sh, gs_v)

        for j in range(CAP // 16):
            idxv = idxl_v[pl.ds(j * 16, 16)]
            gsv = plsc.load_gather(gs_v, [idxv])
            gn_v[pl.ds(j * 16, 16)] = gvals_v[pl.ds(j * 16, 16)] / gsv
        pltpu.sync_copy(idxl_v.at[pl.ds(0, CAP)], tok_hbm.at[e])
        pltpu.sync_copy(gn_v, gn_hbm.at[e])


def _routing(keys_T, probs_T):
    mesh = plsc.VectorSubcoreMesh(**_MESH)
    f = pl.kernel(
        _routing_body,
        out_type=[
            jax.ShapeDtypeStruct((E, CAP), jnp.int32),
            jax.ShapeDtypeStruct((E, CAP), jnp.float32),
        ],
        mesh=mesh,
        compiler_params=pltpu.CompilerParams(needs_layout_passes=False),
        scratch_types=[
            pltpu.VMEM((N,), jnp.int32),
            pltpu.VMEM((N,), jnp.float32),
            pltpu.VMEM((N,), jnp.float32),
            pltpu.VMEM((E, KV), jnp.float32),
            pltpu.VMEM((KV,), jnp.float32),
            pltpu.VMEM((N,), jnp.float32),
            pltpu.VMEM((CAP + 16,), jnp.int32),
            pltpu.VMEM((CAP + 16,), jnp.float32),
            pltpu.VMEM((CAP,), jnp.float32),
            pltpu.VMEM_SHARED((E, N), jnp.float32),
            pltpu.VMEM_SHARED((N,), jnp.float32),
        ],
    )
    return f(keys_T, probs_T)


GCH = 64


def _gather(xn, tok_g, rg):
    rpw = rg // NW

    def body(xn_hbm, idx_hbm, xg_hbm, idx_v, rows_v, sem):
        wid = lax.axis_index("s") * NC + lax.axis_index("c")
        base = wid * rpw
        pltpu.sync_copy(idx_hbm.at[pl.ds(base, rpw)], idx_v)
        for q in range(rpw // GCH):
            pltpu.async_copy(xn_hbm.at[idx_v.at[pl.ds(q * GCH, GCH)]],
                             rows_v, sem).wait()
            pltpu.sync_copy(rows_v, xg_hbm.at[pl.ds(base + q * GCH, GCH)])

    f = pl.kernel(
        body,
        out_type=jax.ShapeDtypeStruct((rg, D2), jnp.int32),
        mesh=plsc.VectorSubcoreMesh(**_MESH),
        compiler_params=pltpu.CompilerParams(needs_layout_passes=False),
        scratch_types=[
            pltpu.VMEM((rpw,), jnp.int32),
            pltpu.VMEM((GCH, D2), jnp.int32),
            pltpu.SemaphoreType.DMA,
        ],
    )
    return f(xn, tok_g)


OSR = N // NS
CBLK = 128


def _scatter(yg_flat, tok3, base, rg):
    asr = rg // NS
    nq = asr // 128

    def body(yg_hbm, idx3_hbm, base_hbm, out_hbm, idx_v, rows_v, acc_sh):
        cid = lax.axis_index("c")
        sid = lax.axis_index("s")
        pltpu.sync_copy(idx3_hbm.at[sid], idx_v)
        for p in range(D // (NC * CBLK)):
            c0 = cid * (D // NC) + p * CBLK
            pltpu.sync_copy(
                base_hbm.at[pl.ds(sid * OSR, OSR), pl.ds(c0, CBLK)],
                acc_sh.at[pl.ds(sid * OSR, OSR)])
            plsc.subcore_barrier()
            pltpu.sync_copy(yg_hbm.at[pl.ds(sid * asr, asr), pl.ds(c0, CBLK)],
                            rows_v)
            for q in range(nq):
                pltpu.sync_copy(rows_v.at[pl.ds(q * 128, 128)],
                                acc_sh.at[idx_v.at[q]], add=True)
            plsc.subcore_barrier()
            pltpu.sync_copy(acc_sh.at[pl.ds(sid * OSR, OSR)],
                            out_hbm.at[pl.ds(sid * OSR, OSR), pl.ds(c0, CBLK)])

    f = pl.kernel(
        body,
        out_type=jax.ShapeDtypeStruct((N, D), jnp.float32),
        mesh=plsc.VectorSubcoreMesh(**_MESH),
        compiler_params=pltpu.CompilerParams(needs_layout_passes=False),
        scratch_types=[
            pltpu.VMEM((nq, 128), jnp.int32),
            pltpu.VMEM((asr, CBLK), jnp.float32),
            pltpu.VMEM_SHARED((N, CBLK), jnp.float32),
        ],
    )
    return f(yg_flat, tok3, base)


def kernel(x, norm_weight, router_weight, router_bias, fc1_weight, fc1_bias,
           fc2_weight, fc2_bias):
    x_flat = x.reshape(N, D)
    xn, keys_T, probs_T = _router(x_flat, norm_weight, router_weight,
                                  router_bias)
    token_idx, g_norm = _routing(keys_T, probs_T)
    tok_flat = token_idx.reshape(N)
    xg = _gather(xn, tok_flat, N)
    yg = _mlp(xg, g_norm, fc1_weight, fc1_bias, fc2_weight, fc2_bias, 0, E)
    out = _scatter(yg.reshape(N, D), token_idx.reshape(NS, N // NS // 128, 128),
                   jnp.zeros((N, D), jnp.float32), N)
    return out.reshape(B, L, D)


# speedup vs baseline: 1.6695x; 1.0034x over previous
import functools
import math

import jax
import jax.numpy as jnp
from jax import lax
from jax.experimental import pallas as pl
from jax.experimental.pallas import tpu as pltpu
from jax.experimental.pallas import tpu_sc as plsc

B, L, D, E, FF = 2, 2048, 1024, 16, 4096
N = B * L
CAP = N // E
D2 = D // 2
EPS = 1e-05

TOK_BLK = 512
FF_BLK = 2048


def _router_body(x_ref, nw_ref, rw_ref, rb_ref, xn_ref, kt_ref, pt_ref):
    x = x_ref[...]
    var = jnp.mean(x * x, axis=1, keepdims=True)
    xn = x * lax.rsqrt(var + EPS) * nw_ref[...]
    xnb = xn.astype(jnp.bfloat16)
    lo = lax.bitcast_convert_type(xnb[:, :D2], jnp.int16).astype(jnp.int32)
    hi = lax.bitcast_convert_type(xnb[:, D2:], jnp.int16).astype(jnp.int32)
    xn_ref[...] = jnp.left_shift(hi, 16) | (lo & 0xFFFF)
    lt = lax.dot_general(rw_ref[...], xn, (((1,), (1,)), ((), ())),
                         preferred_element_type=jnp.float32)
    lt = lt + rb_ref[...].reshape(E, 1)
    kb = lax.bitcast_convert_type(lt, jnp.int32)
    kt_ref[...] = jnp.where(kb < 0, kb ^ jnp.int32(0x7FFFFFFF), kb)
    m = jnp.max(lt, axis=0, keepdims=True)
    ex = jnp.exp(lt - m)
    pt_ref[...] = ex / jnp.sum(ex, axis=0, keepdims=True)


def _router(x_flat, norm_weight, router_weight, router_bias):
    grid = (N // TOK_BLK,)
    return pl.pallas_call(
        _router_body,
        grid=grid,
        in_specs=[
            pl.BlockSpec((TOK_BLK, D), lambda i: (i, 0)),
            pl.BlockSpec((1, D), lambda i: (0, 0)),
            pl.BlockSpec((E, D), lambda i: (0, 0)),
            pl.BlockSpec((1, E), lambda i: (0, 0)),
        ],
        out_specs=[
            pl.BlockSpec((TOK_BLK, D2), lambda i: (i, 0)),
            pl.BlockSpec((E, TOK_BLK), lambda i: (0, i)),
            pl.BlockSpec((E, TOK_BLK), lambda i: (0, i)),
        ],
        out_shape=[
            jax.ShapeDtypeStruct((N, D2), jnp.int32),
            jax.ShapeDtypeStruct((E, N), jnp.int32),
            jax.ShapeDtypeStruct((E, N), jnp.float32),
        ],
    )(x_flat, norm_weight.reshape(1, D), router_weight, router_bias.reshape(1, E))


def _gelu_exact(h):
    return 0.5 * h * (1.0 + lax.erf(h * (1.0 / math.sqrt(2.0))))


def _mlp_body(xg_ref, g_ref, w1_ref, b1_ref, w2_ref, b2_ref, out_ref, acc_ref):
    f = pl.program_id(1)
    xgi = xg_ref[0]
    xlo = lax.bitcast_convert_type(xgi.astype(jnp.int16), jnp.bfloat16)
    xhi = lax.bitcast_convert_type(
        jnp.right_shift(xgi, 16).astype(jnp.int16), jnp.bfloat16)
    xg = jnp.concatenate([xlo, xhi], axis=1)
    w1 = w1_ref[0].astype(jnp.bfloat16)
    h = lax.dot_general(xg, w1, (((1,), (1,)), ((), ())),
                        preferred_element_type=jnp.float32)
    h = h + b1_ref[0]
    h = _gelu_exact(h)
    w2 = w2_ref[0].astype(jnp.bfloat16)
    y = lax.dot_general(h.astype(jnp.bfloat16), w2, (((1,), (1,)), ((), ())),
                        preferred_element_type=jnp.float32)

    @pl.when(f == 0)
    def _():
        acc_ref[...] = y + b2_ref[0]

    @pl.when(f > 0)
    def _():
        acc_ref[...] += y

    @pl.when(f == FF // FF_BLK - 1)
    def _():
        r = acc_ref[...] * g_ref[0].reshape(CAP, 1)
        for c in range(D // CBLK):
            out_ref[c] = r[:, c * CBLK:(c + 1) * CBLK]


def _mlp(xg_g, g_norm, fc1_weight, fc1_bias, fc2_weight, fc2_bias, e0, eg):
    grid = (eg, FF // FF_BLK)
    return pl.pallas_call(
        _mlp_body,
        grid=grid,
        in_specs=[
            pl.BlockSpec((1, CAP, D2), lambda e, f: (e, 0, 0)),
            pl.BlockSpec((1, 1, CAP), lambda e, f: (e + e0, 0, 0)),
            pl.BlockSpec((1, FF_BLK, D), lambda e, f: (e + e0, f, 0)),
            pl.BlockSpec((1, 1, FF_BLK), lambda e, f: (e + e0, 0, f)),
            pl.BlockSpec((1, D, FF_BLK), lambda e, f: (e + e0, 0, f)),
            pl.BlockSpec((1, 1, D), lambda e, f: (e + e0, 0, 0)),
        ],
        out_specs=pl.BlockSpec((D // CBLK, CAP, CBLK), lambda e, f: (0, e, 0)),
        out_shape=jax.ShapeDtypeStruct((D // CBLK, eg * CAP, CBLK),
                                       jnp.float32),
        scratch_shapes=[pltpu.VMEM((CAP, D), jnp.float32)],
        compiler_params=pltpu.CompilerParams(
            dimension_semantics=("parallel", "arbitrary")),
    )(xg_g.reshape(eg, CAP, D2), g_norm.reshape(E, 1, CAP),
      fc1_weight, fc1_bias.reshape(E, 1, FF),
      fc2_weight, fc2_bias.reshape(E, 1, D))


NC, NS = 2, 16
NW = NC * NS
KV = N // 16
_MESH = dict(core_axis_name="c", subcore_axis_name="s")
IMIN = -2147483648


def _routing_body(kt_hbm, pt_hbm, tok_hbm, gn_hbm,
                  keys_v, probs_v, gd_v, seg_v, gsp_v, gs_v,
                  idxl_v, gvals_v, gn_v, gs_all_sh, gs_tot_sh):
    cid = lax.axis_index("c")
    sid = lax.axis_index("s")

    @pl.when(cid == 0)
    def _():
        e = sid
        pltpu.sync_copy(kt_hbm.at[e], keys_v)
        pltpu.sync_copy(pt_hbm.at[e], probs_v)

        def lane_sum(v):
            s = v[0]
            for l in range(1, 16):
                s = s + v[l]
            return s

        def popcnt(m):
            return lane_sum(jnp.where(m, jnp.int32(1), jnp.int32(0)))

        def count_ge(ts):
            def cbody(i, acc):
                k = keys_v[pl.ds(i * 16, 16)]
                return acc + jnp.where(k >= ts, jnp.int32(1), jnp.int32(0))
            acc = lax.fori_loop(0, KV, cbody, jnp.zeros((16,), jnp.int32),
                                unroll=8)
            return lane_sum(acc)

        def bbody(b, upref):
            cand_u = upref | jnp.left_shift(jnp.int32(1), 31 - b)
            cnt = count_ge(cand_u ^ IMIN)
            return jnp.where(cnt >= CAP, cand_u, upref)
        upref = lax.fori_loop(0, 32, bbody, jnp.int32(0))
        thr = upref ^ IMIN

        def sbody(i, off):
            k = keys_v[pl.ds(i * 16, 16)]
            p = probs_v[pl.ds(i * 16, 16)]
            gt = k > thr
            ids = lax.iota(jnp.int32, 16) + i * 16
            plsc.store_compressed(idxl_v.at[pl.ds(off, 16)], ids, mask=gt)
            plsc.store_compressed(gvals_v.at[pl.ds(off, 16)], p, mask=gt)
            return off + popcnt(gt)
        off = lax.fori_loop(0, KV, sbody, jnp.int32(0), unroll=4)

        def tcond(c):
            i, o = c
            return (i < KV) & (o < CAP)

        def tbody(c):
            i, o = c
            k = keys_v[pl.ds(i * 16, 16)]
            p = probs_v[pl.ds(i * 16, 16)]
            eq = k == thr
            ids = lax.iota(jnp.int32, 16) + i * 16
            plsc.store_compressed(idxl_v.at[pl.ds(o, 16)], ids, mask=eq)
            plsc.store_compressed(gvals_v.at[pl.ds(o, 16)], p, mask=eq)
            return i + 1, o + popcnt(eq)
        lax.while_loop(tcond, tbody, (jnp.int32(0), off))

        def zbody(i, c):
            gd_v[pl.ds(i * 16, 16)] = jnp.zeros((16,), jnp.float32)
            return c
        lax.fori_loop(0, KV, zbody, jnp.int32(0), unroll=8)
        for j in range(CAP // 16):
            idxv = idxl_v[pl.ds(j * 16, 16)]
            plsc.store_scatter(gd_v, [idxv], gvals_v[pl.ds(j * 16, 16)])

        pltpu.sync_copy(gd_v, gs_all_sh.at[e])
        plsc.subcore_barrier()
        pltpu.sync_copy(gs_all_sh.at[:, pl.ds(sid * KV, KV)], seg_v)
        for c in range(KV // 16):
            acc = seg_v[0, pl.ds(c * 16, 16)]
            for ee in range(1, E):
                acc = acc + seg_v[ee, pl.ds(c * 16, 16)]
            gsp_v[pl.ds(c * 16, 16)] = acc
        pltpu.sync_copy(gsp_v, gs_tot_sh.at[pl.ds(sid * KV, KV)])
        plsc.subcore_barrier()
        pltpu.sync_copy(gs_tot_sh, gs_v)

        for j in range(CAP // 16):
            idxv = idxl_v[pl.ds(j * 16, 16)]
            gsv = plsc.load_gather(gs_v, [idxv])
            gn_v[pl.ds(j * 16, 16)] = gvals_v[pl.ds(j * 16, 16)] / gsv
        pltpu.sync_copy(idxl_v.at[pl.ds(0, CAP)], tok_hbm.at[e])
        pltpu.sync_copy(gn_v, gn_hbm.at[e])


def _routing(keys_T, probs_T):
    mesh = plsc.VectorSubcoreMesh(**_MESH)
    f = pl.kernel(
        _routing_body,
        out_type=[
            jax.ShapeDtypeStruct((E, CAP), jnp.int32),
            jax.ShapeDtypeStruct((E, CAP), jnp.float32),
        ],
        mesh=mesh,
        compiler_params=pltpu.CompilerParams(needs_layout_passes=False),
        scratch_types=[
            pltpu.VMEM((N,), jnp.int32),
            pltpu.VMEM((N,), jnp.float32),
            pltpu.VMEM((N,), jnp.float32),
            pltpu.VMEM((E, KV), jnp.float32),
            pltpu.VMEM((KV,), jnp.float32),
            pltpu.VMEM((N,), jnp.float32),
            pltpu.VMEM((CAP + 16,), jnp.int32),
            pltpu.VMEM((CAP + 16,), jnp.float32),
            pltpu.VMEM((CAP,), jnp.float32),
            pltpu.VMEM_SHARED((E, N), jnp.float32),
            pltpu.VMEM_SHARED((N,), jnp.float32),
        ],
    )
    return f(keys_T, probs_T)


GCH = 64


def _gather(xn, tok_g, rg):
    rpw = rg // NW

    def body(xn_hbm, idx_hbm, xg_hbm, idx_v, rows_v, sem):
        wid = lax.axis_index("s") * NC + lax.axis_index("c")
        base = wid * rpw
        pltpu.sync_copy(idx_hbm.at[pl.ds(base, rpw)], idx_v)
        for q in range(rpw // GCH):
            pltpu.async_copy(xn_hbm.at[idx_v.at[pl.ds(q * GCH, GCH)]],
                             rows_v, sem).wait()
            pltpu.sync_copy(rows_v, xg_hbm.at[pl.ds(base + q * GCH, GCH)])

    f = pl.kernel(
        body,
        out_type=jax.ShapeDtypeStruct((rg, D2), jnp.int32),
        mesh=plsc.VectorSubcoreMesh(**_MESH),
        compiler_params=pltpu.CompilerParams(needs_layout_passes=False),
        scratch_types=[
            pltpu.VMEM((rpw,), jnp.int32),
            pltpu.VMEM((GCH, D2), jnp.int32),
            pltpu.SemaphoreType.DMA,
        ],
    )
    return f(xn, tok_g)


OSR = N // NS
CBLK = 128


def _scatter(yg_t, tok3, base, rg):
    asr = rg // NS
    nq = asr // 128

    def body(yg_hbm, idx3_hbm, base_hbm, out_hbm, idx_v, rows_v, acc_sh):
        cid = lax.axis_index("c")
        sid = lax.axis_index("s")
        pltpu.sync_copy(idx3_hbm.at[sid], idx_v)
        for p in range(D // (NC * CBLK)):
            c0 = cid * (D // NC) + p * CBLK
            cb = c0 // CBLK
            pltpu.sync_copy(
                base_hbm.at[pl.ds(sid * OSR, OSR), pl.ds(c0, CBLK)],
                acc_sh.at[pl.ds(sid * OSR, OSR)])
            plsc.subcore_barrier()
            pltpu.sync_copy(yg_hbm.at[cb, pl.ds(sid * asr, asr)], rows_v)
            for q in range(nq):
                pltpu.sync_copy(rows_v.at[pl.ds(q * 128, 128)],
                                acc_sh.at[idx_v.at[q]], add=True)
            plsc.subcore_barrier()
            pltpu.sync_copy(acc_sh.at[pl.ds(sid * OSR, OSR)],
                            out_hbm.at[pl.ds(sid * OSR, OSR), pl.ds(c0, CBLK)])

    f = pl.kernel(
        body,
        out_type=jax.ShapeDtypeStruct((N, D), jnp.float32),
        mesh=plsc.VectorSubcoreMesh(**_MESH),
        compiler_params=pltpu.CompilerParams(needs_layout_passes=False),
        scratch_types=[
            pltpu.VMEM((nq, 128), jnp.int32),
            pltpu.VMEM((asr, CBLK), jnp.float32),
            pltpu.VMEM_SHARED((N, CBLK), jnp.float32),
        ],
    )
    return f(yg_t, tok3, base)


def kernel(x, norm_weight, router_weight, router_bias, fc1_weight, fc1_bias,
           fc2_weight, fc2_bias):
    x_flat = x.reshape(N, D)
    xn, keys_T, probs_T = _router(x_flat, norm_weight, router_weight,
                                  router_bias)
    token_idx, g_norm = _routing(keys_T, probs_T)
    tok_flat = token_idx.reshape(N)
    xg = _gather(xn, tok_flat, N)
    yg_t = _mlp(xg, g_norm, fc1_weight, fc1_bias, fc2_weight, fc2_bias, 0, E)
    out = _scatter(yg_t, token_idx.reshape(NS, N // NS // 128, 128),
                   jnp.zeros((N, D), jnp.float32), N)
    return out.reshape(B, L, D)
